# baseline (speedup 1.0000x reference)
import functools

import jax
import jax.numpy as jnp
from jax import lax
from jax.experimental import pallas as pl
from jax.experimental.pallas import tpu as pltpu
from jax.experimental.pallas import tpu_sc as plsc

DIM = 1024
HIDDEN = 1024
E = 8
TOP_K = 2
NT = 2048
LANES = 128
NEG = float("-inf")

BLK = 512
NBLK = NT * TOP_K // BLK + (E - 1)
RPAD = NBLK * BLK

NWORKERS = 32
TPW = NT // NWORKERS
SL = DIM // LANES

_DN = (((1,), (1,)), ((), ()))


def _router_body(xf_ref, g_ref, b_ref, d0_ref, d1_ref, s0_ref,
                 s1_ref, be_ref, act_ref):
    xf = xf_ref[...]
    scores = jnp.dot(xf, g_ref[...], preferred_element_type=jnp.float32)
    sig = jax.nn.sigmoid(scores)
    lane = lax.broadcasted_iota(jnp.int32, (NT, E), 1)
    biased = sig + b_ref[0, :][None, :]
    m1 = jnp.max(biased, axis=1, keepdims=True)
    idx1 = jnp.min(jnp.where(biased == m1, lane, E), axis=1, keepdims=True)
    sel1 = lane == idx1
    b2 = jnp.where(sel1, NEG, biased)
    m2 = jnp.max(b2, axis=1, keepdims=True)
    idx2 = jnp.min(jnp.where(b2 == m2, lane, E), axis=1, keepdims=True)
    sel2 = lane == idx2
    s0_ref[...] = jnp.sum(jnp.where(sel1, sig, 0.0), axis=1, keepdims=True)
    s1_ref[...] = jnp.sum(jnp.where(sel2, sig, 0.0), axis=1, keepdims=True)

    c = jnp.where(sel1 | sel2, 1.0, 0.0)
    ii = lax.broadcasted_iota(jnp.int32, (LANES, LANES), 0)
    jj = lax.broadcasted_iota(jnp.int32, (LANES, LANES), 1)
    lt = jnp.where(jj < ii, 1.0, 0.0)
    nch = NT // LANES
    base = jnp.zeros((1, E), jnp.float32)
    excl_chunks = []
    for g in range(nch):
        cg = lax.slice(c, (g * LANES, 0), ((g + 1) * LANES, E))
        eg = jnp.dot(lt, cg, preferred_element_type=jnp.float32)
        excl_chunks.append(eg + base)
        base = base + jnp.sum(cg, axis=0, keepdims=True)
    excl = jnp.concatenate(excl_chunks, axis=0)
    counts = base

    pc = jnp.floor((counts + (BLK - 1)) * (1.0 / BLK)) * BLK
    ii8 = lax.broadcasted_iota(jnp.int32, (E, E), 0)
    jj8 = lax.broadcasted_iota(jnp.int32, (E, E), 1)
    su = jnp.where(ii8 < jj8, 1.0, 0.0)
    offs = jnp.dot(pc, su, preferred_element_type=jnp.float32)
    dest = offs + excl
    d0_ref[...] = jnp.sum(jnp.where(sel1, dest, 0.0), axis=1,
                          keepdims=True).astype(jnp.int32)
    d1_ref[...] = jnp.sum(jnp.where(sel2, dest, 0.0), axis=1,
                          keepdims=True).astype(jnp.int32)

    cum_incl = offs + pc
    bcol = lax.broadcasted_iota(jnp.int32, (LANES, 1), 0).astype(jnp.float32)
    cmp = jnp.where(cum_incl * (1.0 / BLK) <= bcol, 1.0, 0.0)
    be = jnp.minimum(jnp.sum(cmp, axis=1, keepdims=True), float(E - 1))
    be_ref[...] = be.astype(jnp.int32)
    tp = jnp.sum(pc, axis=1, keepdims=True)
    act_ref[...] = jnp.where(bcol * BLK < tp, 1, 0).astype(jnp.int32)


def _run_router(xf, gate, expert_bias):
    bp = expert_bias.reshape(1, E)
    outs = pl.pallas_call(
        _router_body,
        grid=(1,),
        in_specs=[
            pl.BlockSpec((NT, DIM), lambda i: (0, 0)),
            pl.BlockSpec((DIM, E), lambda i: (0, 0)),
            pl.BlockSpec((1, E), lambda i: (0, 0)),
        ],
        out_specs=[
            pl.BlockSpec((NT, 1), lambda i: (0, 0)),
            pl.BlockSpec((NT, 1), lambda i: (0, 0)),
            pl.BlockSpec((NT, 1), lambda i: (0, 0)),
            pl.BlockSpec((NT, 1), lambda i: (0, 0)),
            pl.BlockSpec((LANES, 1), lambda i: (0, 0)),
            pl.BlockSpec((LANES, 1), lambda i: (0, 0)),
        ],
        out_shape=[
            jax.ShapeDtypeStruct((NT, 1), jnp.int32),
            jax.ShapeDtypeStruct((NT, 1), jnp.int32),
            jax.ShapeDtypeStruct((NT, 1), jnp.float32),
            jax.ShapeDtypeStruct((NT, 1), jnp.float32),
            jax.ShapeDtypeStruct((LANES, 1), jnp.int32),
            jax.ShapeDtypeStruct((LANES, 1), jnp.int32),
        ],
    )(xf, gate, bp)
    return outs


def _dispatch_body(x_hbm, d0_hbm, d1_hbm, xg_hbm, rows_v, d0_v, d1_v, sem):
    wid = lax.axis_index("s") * 2 + lax.axis_index("c")
    base = wid * TPW
    l0 = pltpu.async_copy(d0_hbm.at[pl.ds(base, TPW)], d0_v, sem)
    l1 = pltpu.async_copy(d1_hbm.at[pl.ds(base, TPW)], d1_v, sem)
    l2 = pltpu.async_copy(x_hbm.at[pl.ds(base, TPW)], rows_v, sem)
    l0.wait()
    l1.wait()
    l2.wait()
    c0 = pltpu.async_copy(rows_v, xg_hbm.at[d0_v], sem)
    c1 = pltpu.async_copy(rows_v, xg_hbm.at[d1_v], sem)
    c0.wait()
    c1.wait()


@functools.cache
def _build_dispatch():
    return pl.kernel(
        _dispatch_body,
        out_type=jax.ShapeDtypeStruct((RPAD, DIM), jnp.float32),
        mesh=plsc.VectorSubcoreMesh(core_axis_name="c", subcore_axis_name="s"),
        scratch_types=[
            pltpu.VMEM((TPW, DIM), jnp.float32),
            pltpu.VMEM((TPW,), jnp.int32),
            pltpu.VMEM((TPW,), jnp.int32),
            pltpu.SemaphoreType.DMA,
        ],
    )


def _dispatch(xf, d0, d1):
    return _build_dispatch()(xf, d0, d1)


def _gmm_body(be_ref, act_ref, na_ref, xg_ref, w1_ref, w3_ref, w2_ref,
              eo_ref, w1s, w3s, w2s):
    b = pl.program_id(0)
    prev = be_ref[jnp.maximum(b - 1, 0)]
    changed = (b == 0) | (be_ref[b] != prev)

    @pl.when((act_ref[b] == 1) & changed)
    def _():
        w1s[...] = w1_ref[0].astype(jnp.bfloat16)
        w3s[...] = w3_ref[0].astype(jnp.bfloat16)
        w2s[...] = w2_ref[0].astype(jnp.bfloat16)

    @pl.when(act_ref[b] == 1)
    def _():
        xb = xg_ref[...].astype(jnp.bfloat16)
        t1 = lax.dot_general(xb, w1s[...], _DN,
                             preferred_element_type=jnp.float32)
        t3 = lax.dot_general(xb, w3s[...], _DN,
                             preferred_element_type=jnp.float32)
        h = ((t1 * jax.nn.sigmoid(t1)) * t3).astype(jnp.bfloat16)
        eo_ref[...] = lax.dot_general(
            h, w2s[...], _DN, preferred_element_type=jnp.float32
        ).astype(jnp.bfloat16).astype(jnp.float32)


def _run_gmm(be, act, na, xg, w1, w3, w2):
    return pl.pallas_call(
        _gmm_body,
        grid_spec=pltpu.PrefetchScalarGridSpec(
            num_scalar_prefetch=3,
            grid=(NBLK,),
            in_specs=[
                pl.BlockSpec(
                    (BLK, DIM),
                    lambda b, be_r, act_r, na_r:
                        (jnp.minimum(b, na_r[0] - 1), 0)),
                pl.BlockSpec((1, HIDDEN, DIM),
                             lambda b, be_r, act_r, na_r: (be_r[b], 0, 0)),
                pl.BlockSpec((1, HIDDEN, DIM),
                             lambda b, be_r, act_r, na_r: (be_r[b], 0, 0)),
                pl.BlockSpec((1, DIM, HIDDEN),
                             lambda b, be_r, act_r, na_r: (be_r[b], 0, 0)),
            ],
            out_specs=pl.BlockSpec(
                (BLK, DIM),
                lambda b, be_r, act_r, na_r:
                    (jnp.minimum(b, na_r[0] - 1), 0)),
            scratch_shapes=[
                pltpu.VMEM((HIDDEN, DIM), jnp.bfloat16),
                pltpu.VMEM((HIDDEN, DIM), jnp.bfloat16),
                pltpu.VMEM((DIM, HIDDEN), jnp.bfloat16),
            ],
        ),
        out_shape=jax.ShapeDtypeStruct((RPAD, DIM), jnp.float32),
        compiler_params=pltpu.CompilerParams(
            vmem_limit_bytes=100 * 1024 * 1024,
        ),
    )(be, act, na, xg, w1, w3, w2)


def _collect_body(eo_hbm, d0_hbm, d1_hbm, g0_hbm, g1_hbm,
                  r0_v, r1_v, d0_v, d1_v, sem):
    wid = lax.axis_index("s") * 2 + lax.axis_index("c")
    base = wid * TPW
    half = TPW // 2
    l0 = pltpu.async_copy(d0_hbm.at[pl.ds(base, TPW)], d0_v, sem)
    l1 = pltpu.async_copy(d1_hbm.at[pl.ds(base, TPW)], d1_v, sem)
    l0.wait()
    l1.wait()
    for p in range(2):
        off = p * half
        g0 = pltpu.async_copy(eo_hbm.at[d0_v.at[pl.ds(off, half)]], r0_v, sem)
        g1 = pltpu.async_copy(eo_hbm.at[d1_v.at[pl.ds(off, half)]], r1_v, sem)
        g0.wait()
        g1.wait()
        s0 = pltpu.async_copy(r0_v, g0_hbm.at[pl.ds(base + off, half)], sem)
        s1 = pltpu.async_copy(r1_v, g1_hbm.at[pl.ds(base + off, half)], sem)
        s0.wait()
        s1.wait()


@functools.cache
def _build_collect():
    return pl.kernel(
        _collect_body,
        out_type=(jax.ShapeDtypeStruct((NT, DIM), jnp.float32),
                  jax.ShapeDtypeStruct((NT, DIM), jnp.float32)),
        mesh=plsc.VectorSubcoreMesh(core_axis_name="c", subcore_axis_name="s"),
        scratch_types=[
            pltpu.VMEM((TPW // 2, DIM), jnp.float32),
            pltpu.VMEM((TPW // 2, DIM), jnp.float32),
            pltpu.VMEM((TPW,), jnp.int32),
            pltpu.VMEM((TPW,), jnp.int32),
            pltpu.SemaphoreType.DMA,
        ],
    )


def _collect(eo, d0, d1):
    return _build_collect()(eo, d0, d1)


def _shared_body(xf_ref, ws1_ref, ws2_ref, ws3_ref, sh_ref, ws1s, ws2s, ws3s):
    i = pl.program_id(0)

    @pl.when(i == 0)
    def _():
        ws1s[...] = ws1_ref[...].astype(jnp.bfloat16)
        ws2s[...] = ws2_ref[...].astype(jnp.bfloat16)
        ws3s[...] = ws3_ref[...].astype(jnp.bfloat16)

    xb = xf_ref[...].astype(jnp.bfloat16)
    u1 = lax.dot_general(xb, ws1s[...], _DN,
                         preferred_element_type=jnp.float32)
    u3 = lax.dot_general(xb, ws3s[...], _DN,
                         preferred_element_type=jnp.float32)
    hs = ((u1 * jax.nn.sigmoid(u1)) * u3).astype(jnp.bfloat16)
    sh_ref[...] = lax.dot_general(hs, ws2s[...], _DN,
                                  preferred_element_type=jnp.float32)


def _run_shared(xf, ws1, ws2, ws3):
    nb = 2
    bt = NT // nb
    return pl.pallas_call(
        _shared_body,
        grid=(nb,),
        in_specs=[
            pl.BlockSpec((bt, DIM), lambda i: (i, 0)),
            pl.BlockSpec((HIDDEN, DIM), lambda i: (0, 0)),
            pl.BlockSpec((DIM, HIDDEN), lambda i: (0, 0)),
            pl.BlockSpec((HIDDEN, DIM), lambda i: (0, 0)),
        ],
        out_specs=pl.BlockSpec((bt, DIM), lambda i: (i, 0)),
        out_shape=jax.ShapeDtypeStruct((NT, DIM), jnp.float32),
        scratch_shapes=[
            pltpu.VMEM((HIDDEN, DIM), jnp.bfloat16),
            pltpu.VMEM((DIM, HIDDEN), jnp.bfloat16),
            pltpu.VMEM((HIDDEN, DIM), jnp.bfloat16),
        ],
    )(xf, ws1, ws2, ws3)


def _combine_body(sh_ref, g0_ref, g1_ref, s0_ref, s1_ref, out_ref):
    out_ref[...] = (s0_ref[...] * g0_ref[...]
                    + s1_ref[...] * g1_ref[...] + sh_ref[...])


def _run_combine(sh, g0, g1, s0, s1):
    nb = 4
    bt = NT // nb
    return pl.pallas_call(
        _combine_body,
        grid=(nb,),
        in_specs=[
            pl.BlockSpec((bt, DIM), lambda i: (i, 0)),
            pl.BlockSpec((bt, DIM), lambda i: (i, 0)),
            pl.BlockSpec((bt, DIM), lambda i: (i, 0)),
            pl.BlockSpec((bt, 1), lambda i: (i, 0)),
            pl.BlockSpec((bt, 1), lambda i: (i, 0)),
        ],
        out_specs=pl.BlockSpec((bt, DIM), lambda i: (i, 0)),
        out_shape=jax.ShapeDtypeStruct((NT, DIM), jnp.float32),
    )(sh, g0, g1, s0, s1)


def kernel(x, gate, w1, w2, w3, ws1, ws2, ws3, expert_bias):
    ob, ib, slen, dim = x.shape
    xf = x.reshape(NT, DIM)

    d0_2d, d1_2d, s0, s1, be_2d, act_2d = _run_router(xf, gate, expert_bias)
    d0 = d0_2d.reshape(NT)
    d1 = d1_2d.reshape(NT)
    be = be_2d.reshape(LANES)[:NBLK]
    act = act_2d.reshape(LANES)[:NBLK]

    xg = _dispatch(xf, d0, d1)

    na = jnp.maximum(jnp.sum(act), 1)
    eo = _run_gmm(be, act, na.reshape(1), xg, w1, w3, w2)

    sh = _run_shared(xf, ws1, ws2, ws3)
    g0, g1 = _collect(eo, d0, d1)

    out = _run_combine(sh, g0, g1, s0, s1)
    return out.reshape(ob, ib, slen, dim)


# speedup vs baseline: 1.0088x; 1.0088x over previous
import functools

import jax
import jax.numpy as jnp
from jax import lax
from jax.experimental import pallas as pl
from jax.experimental.pallas import tpu as pltpu
from jax.experimental.pallas import tpu_sc as plsc

DIM = 1024
HIDDEN = 1024
E = 8
TOP_K = 2
NT = 2048
LANES = 128
NEG = float("-inf")

BLK = 512
NBLK = NT * TOP_K // BLK + (E - 1)
RPAD = NBLK * BLK

NWORKERS = 32
TPW = NT // NWORKERS
SL = DIM // LANES

_DN = (((1,), (1,)), ((), ()))


def _router_body(xf_ref, g_ref, b_ref, d0_ref, d1_ref, s0_ref,
                 s1_ref, be_ref, act_ref):
    xf = xf_ref[...]
    scores = jnp.dot(xf, g_ref[...], preferred_element_type=jnp.float32)
    sig = jax.nn.sigmoid(scores)
    lane = lax.broadcasted_iota(jnp.int32, (NT, E), 1)
    biased = sig + b_ref[0, :][None, :]
    m1 = jnp.max(biased, axis=1, keepdims=True)
    idx1 = jnp.min(jnp.where(biased == m1, lane, E), axis=1, keepdims=True)
    sel1 = lane == idx1
    b2 = jnp.where(sel1, NEG, biased)
    m2 = jnp.max(b2, axis=1, keepdims=True)
    idx2 = jnp.min(jnp.where(b2 == m2, lane, E), axis=1, keepdims=True)
    sel2 = lane == idx2
    s0_ref[...] = jnp.sum(jnp.where(sel1, sig, 0.0), axis=1, keepdims=True)
    s1_ref[...] = jnp.sum(jnp.where(sel2, sig, 0.0), axis=1, keepdims=True)

    c = jnp.where(sel1 | sel2, 1.0, 0.0)
    ii = lax.broadcasted_iota(jnp.int32, (LANES, LANES), 0)
    jj = lax.broadcasted_iota(jnp.int32, (LANES, LANES), 1)
    lt = jnp.where(jj < ii, 1.0, 0.0)
    nch = NT // LANES
    base = jnp.zeros((1, E), jnp.float32)
    excl_chunks = []
    for g in range(nch):
        cg = lax.slice(c, (g * LANES, 0), ((g + 1) * LANES, E))
        eg = jnp.dot(lt, cg, preferred_element_type=jnp.float32)
        excl_chunks.append(eg + base)
        base = base + jnp.sum(cg, axis=0, keepdims=True)
    excl = jnp.concatenate(excl_chunks, axis=0)
    counts = base

    pc = jnp.floor((counts + (BLK - 1)) * (1.0 / BLK)) * BLK
    ii8 = lax.broadcasted_iota(jnp.int32, (E, E), 0)
    jj8 = lax.broadcasted_iota(jnp.int32, (E, E), 1)
    su = jnp.where(ii8 < jj8, 1.0, 0.0)
    offs = jnp.dot(pc, su, preferred_element_type=jnp.float32)
    dest = offs + excl
    d0_ref[...] = jnp.sum(jnp.where(sel1, dest, 0.0), axis=1,
                          keepdims=True).astype(jnp.int32)
    d1_ref[...] = jnp.sum(jnp.where(sel2, dest, 0.0), axis=1,
                          keepdims=True).astype(jnp.int32)

    cum_incl = offs + pc
    bcol = lax.broadcasted_iota(jnp.int32, (LANES, 1), 0).astype(jnp.float32)
    cmp = jnp.where(cum_incl * (1.0 / BLK) <= bcol, 1.0, 0.0)
    be = jnp.minimum(jnp.sum(cmp, axis=1, keepdims=True), float(E - 1))
    be_ref[...] = be.astype(jnp.int32)
    tp = jnp.sum(pc, axis=1, keepdims=True)
    act_ref[...] = jnp.where(bcol * BLK < tp, 1, 0).astype(jnp.int32)


def _run_router(xf, gate, expert_bias):
    bp = expert_bias.reshape(1, E)
    outs = pl.pallas_call(
        _router_body,
        grid=(1,),
        in_specs=[
            pl.BlockSpec((NT, DIM), lambda i: (0, 0)),
            pl.BlockSpec((DIM, E), lambda i: (0, 0)),
            pl.BlockSpec((1, E), lambda i: (0, 0)),
        ],
        out_specs=[
            pl.BlockSpec((NT, 1), lambda i: (0, 0)),
            pl.BlockSpec((NT, 1), lambda i: (0, 0)),
            pl.BlockSpec((NT, 1), lambda i: (0, 0)),
            pl.BlockSpec((NT, 1), lambda i: (0, 0)),
            pl.BlockSpec((LANES, 1), lambda i: (0, 0)),
            pl.BlockSpec((LANES, 1), lambda i: (0, 0)),
        ],
        out_shape=[
            jax.ShapeDtypeStruct((NT, 1), jnp.int32),
            jax.ShapeDtypeStruct((NT, 1), jnp.int32),
            jax.ShapeDtypeStruct((NT, 1), jnp.float32),
            jax.ShapeDtypeStruct((NT, 1), jnp.float32),
            jax.ShapeDtypeStruct((LANES, 1), jnp.int32),
            jax.ShapeDtypeStruct((LANES, 1), jnp.int32),
        ],
    )(xf, gate, bp)
    return outs


def _dispatch_body(x_hbm, d0_hbm, d1_hbm, xg_hbm, rows_v, d0_v, d1_v, sem):
    wid = lax.axis_index("s") * 2 + lax.axis_index("c")
    base = wid * TPW
    l0 = pltpu.async_copy(d0_hbm.at[pl.ds(base, TPW)], d0_v, sem)
    l1 = pltpu.async_copy(d1_hbm.at[pl.ds(base, TPW)], d1_v, sem)
    l2 = pltpu.async_copy(x_hbm.at[pl.ds(base, TPW)], rows_v, sem)
    l0.wait()
    l1.wait()
    l2.wait()
    c0 = pltpu.async_copy(rows_v, xg_hbm.at[d0_v], sem)
    c1 = pltpu.async_copy(rows_v, xg_hbm.at[d1_v], sem)
    c0.wait()
    c1.wait()


@functools.cache
def _build_dispatch():
    return pl.kernel(
        _dispatch_body,
        out_type=jax.ShapeDtypeStruct((RPAD, DIM), jnp.float32),
        mesh=plsc.VectorSubcoreMesh(core_axis_name="c", subcore_axis_name="s"),
        scratch_types=[
            pltpu.VMEM((TPW, DIM), jnp.float32),
            pltpu.VMEM((TPW,), jnp.int32),
            pltpu.VMEM((TPW,), jnp.int32),
            pltpu.SemaphoreType.DMA,
        ],
    )


def _dispatch(xf, d0, d1):
    return _build_dispatch()(xf, d0, d1)


def _gmm_body(be_ref, act_ref, na_ref, xg_ref, w1_ref, w3_ref, w2_ref,
              eo_ref, w1s, w3s, w2s):
    b = pl.program_id(0)
    prev = be_ref[jnp.maximum(b - 1, 0)]
    changed = (b == 0) | (be_ref[b] != prev)

    @pl.when((act_ref[b] == 1) & changed)
    def _():
        w1s[...] = w1_ref[0].astype(jnp.bfloat16)
        w3s[...] = w3_ref[0].astype(jnp.bfloat16)
        w2s[...] = w2_ref[0].astype(jnp.bfloat16)

    @pl.when(act_ref[b] == 1)
    def _():
        xb = xg_ref[...].astype(jnp.bfloat16)
        t1 = lax.dot_general(xb, w1s[...], _DN,
                             preferred_element_type=jnp.float32)
        t3 = lax.dot_general(xb, w3s[...], _DN,
                             preferred_element_type=jnp.float32)
        h = ((t1 * jax.nn.sigmoid(t1)) * t3).astype(jnp.bfloat16)
        eo_ref[...] = lax.dot_general(
            h, w2s[...], _DN, preferred_element_type=jnp.float32)


def _run_gmm(be, act, na, xg, w1, w3, w2):
    return pl.pallas_call(
        _gmm_body,
        grid_spec=pltpu.PrefetchScalarGridSpec(
            num_scalar_prefetch=3,
            grid=(NBLK,),
            in_specs=[
                pl.BlockSpec(
                    (BLK, DIM),
                    lambda b, be_r, act_r, na_r:
                        (jnp.minimum(b, na_r[0] - 1), 0)),
                pl.BlockSpec((1, HIDDEN, DIM),
                             lambda b, be_r, act_r, na_r: (be_r[b], 0, 0)),
                pl.BlockSpec((1, HIDDEN, DIM),
                             lambda b, be_r, act_r, na_r: (be_r[b], 0, 0)),
                pl.BlockSpec((1, DIM, HIDDEN),
                             lambda b, be_r, act_r, na_r: (be_r[b], 0, 0)),
            ],
            out_specs=pl.BlockSpec(
                (BLK, DIM),
                lambda b, be_r, act_r, na_r:
                    (jnp.minimum(b, na_r[0] - 1), 0)),
            scratch_shapes=[
                pltpu.VMEM((HIDDEN, DIM), jnp.bfloat16),
                pltpu.VMEM((HIDDEN, DIM), jnp.bfloat16),
                pltpu.VMEM((DIM, HIDDEN), jnp.bfloat16),
            ],
        ),
        out_shape=jax.ShapeDtypeStruct((RPAD, DIM), jnp.float32),
        compiler_params=pltpu.CompilerParams(
            vmem_limit_bytes=100 * 1024 * 1024,
        ),
    )(be, act, na, xg, w1, w3, w2)


def _collect_body(eo_hbm, d0_hbm, d1_hbm, g0_hbm, g1_hbm,
                  r0_v, r1_v, d0_v, d1_v, sem):
    wid = lax.axis_index("s") * 2 + lax.axis_index("c")
    base = wid * TPW
    half = TPW // 2
    l0 = pltpu.async_copy(d0_hbm.at[pl.ds(base, TPW)], d0_v, sem)
    l1 = pltpu.async_copy(d1_hbm.at[pl.ds(base, TPW)], d1_v, sem)
    l0.wait()
    l1.wait()
    for p in range(2):
        off = p * half
        g0 = pltpu.async_copy(eo_hbm.at[d0_v.at[pl.ds(off, half)]], r0_v, sem)
        g1 = pltpu.async_copy(eo_hbm.at[d1_v.at[pl.ds(off, half)]], r1_v, sem)
        g0.wait()
        g1.wait()
        s0 = pltpu.async_copy(r0_v, g0_hbm.at[pl.ds(base + off, half)], sem)
        s1 = pltpu.async_copy(r1_v, g1_hbm.at[pl.ds(base + off, half)], sem)
        s0.wait()
        s1.wait()


@functools.cache
def _build_collect():
    return pl.kernel(
        _collect_body,
        out_type=(jax.ShapeDtypeStruct((NT, DIM), jnp.float32),
                  jax.ShapeDtypeStruct((NT, DIM), jnp.float32)),
        mesh=plsc.VectorSubcoreMesh(core_axis_name="c", subcore_axis_name="s"),
        scratch_types=[
            pltpu.VMEM((TPW // 2, DIM), jnp.float32),
            pltpu.VMEM((TPW // 2, DIM), jnp.float32),
            pltpu.VMEM((TPW,), jnp.int32),
            pltpu.VMEM((TPW,), jnp.int32),
            pltpu.SemaphoreType.DMA,
        ],
    )


def _collect(eo, d0, d1):
    return _build_collect()(eo, d0, d1)


def _shared_body(xf_ref, ws1_ref, ws2_ref, ws3_ref, sh_ref, ws1s, ws2s, ws3s):
    i = pl.program_id(0)

    @pl.when(i == 0)
    def _():
        ws1s[...] = ws1_ref[...].astype(jnp.bfloat16)
        ws2s[...] = ws2_ref[...].astype(jnp.bfloat16)
        ws3s[...] = ws3_ref[...].astype(jnp.bfloat16)

    xb = xf_ref[...].astype(jnp.bfloat16)
    u1 = lax.dot_general(xb, ws1s[...], _DN,
                         preferred_element_type=jnp.float32)
    u3 = lax.dot_general(xb, ws3s[...], _DN,
                         preferred_element_type=jnp.float32)
    hs = ((u1 * jax.nn.sigmoid(u1)) * u3).astype(jnp.bfloat16)
    sh_ref[...] = lax.dot_general(hs, ws2s[...], _DN,
                                  preferred_element_type=jnp.float32)


def _run_shared(xf, ws1, ws2, ws3):
    nb = 4
    bt = NT // nb
    return pl.pallas_call(
        _shared_body,
        grid=(nb,),
        in_specs=[
            pl.BlockSpec((bt, DIM), lambda i: (i, 0)),
            pl.BlockSpec((HIDDEN, DIM), lambda i: (0, 0)),
            pl.BlockSpec((DIM, HIDDEN), lambda i: (0, 0)),
            pl.BlockSpec((HIDDEN, DIM), lambda i: (0, 0)),
        ],
        out_specs=pl.BlockSpec((bt, DIM), lambda i: (i, 0)),
        out_shape=jax.ShapeDtypeStruct((NT, DIM), jnp.float32),
        scratch_shapes=[
            pltpu.VMEM((HIDDEN, DIM), jnp.bfloat16),
            pltpu.VMEM((DIM, HIDDEN), jnp.bfloat16),
            pltpu.VMEM((HIDDEN, DIM), jnp.bfloat16),
        ],
    )(xf, ws1, ws2, ws3)


def _combine_body(sh_ref, g0_ref, g1_ref, s0_ref, s1_ref, out_ref):
    out_ref[...] = (s0_ref[...] * g0_ref[...]
                    + s1_ref[...] * g1_ref[...] + sh_ref[...])


def _run_combine(sh, g0, g1, s0, s1):
    nb = 4
    bt = NT // nb
    return pl.pallas_call(
        _combine_body,
        grid=(nb,),
        in_specs=[
            pl.BlockSpec((bt, DIM), lambda i: (i, 0)),
            pl.BlockSpec((bt, DIM), lambda i: (i, 0)),
            pl.BlockSpec((bt, DIM), lambda i: (i, 0)),
            pl.BlockSpec((bt, 1), lambda i: (i, 0)),
            pl.BlockSpec((bt, 1), lambda i: (i, 0)),
        ],
        out_specs=pl.BlockSpec((bt, DIM), lambda i: (i, 0)),
        out_shape=jax.ShapeDtypeStruct((NT, DIM), jnp.float32),
    )(sh, g0, g1, s0, s1)


def kernel(x, gate, w1, w2, w3, ws1, ws2, ws3, expert_bias):
    ob, ib, slen, dim = x.shape
    xf = x.reshape(NT, DIM)

    d0_2d, d1_2d, s0, s1, be_2d, act_2d = _run_router(xf, gate, expert_bias)
    d0 = d0_2d.reshape(NT)
    d1 = d1_2d.reshape(NT)
    be = be_2d.reshape(LANES)[:NBLK]
    act = act_2d.reshape(LANES)[:NBLK]

    xg = _dispatch(xf, d0, d1)

    na = jnp.maximum(jnp.sum(act), 1)
    eo = _run_gmm(be, act, na.reshape(1), xg, w1, w3, w2)

    sh = _run_shared(xf, ws1, ws2, ws3)
    g0, g1 = _collect(eo, d0, d1)

    out = _run_combine(sh, g0, g1, s0, s1)
    return out.reshape(ob, ib, slen, dim)


# speedup vs baseline: 1.0686x; 1.0593x over previous
import functools

import jax
import jax.numpy as jnp
from jax import lax
from jax.experimental import pallas as pl
from jax.experimental.pallas import tpu as pltpu
from jax.experimental.pallas import tpu_sc as plsc

DIM = 1024
HIDDEN = 1024
E = 8
TOP_K = 2
NT = 2048
LANES = 128
NEG = float("-inf")

BLK = 768
NBLK = (NT * TOP_K - E) // BLK + E
RPAD = NBLK * BLK

NWORKERS = 32
TPW = NT // NWORKERS
SL = DIM // LANES

_DN = (((1,), (1,)), ((), ()))


def _router_body(xf_ref, g_ref, b_ref, d0_ref, d1_ref, s0_ref,
                 s1_ref, be_ref, act_ref):
    xf = xf_ref[...]
    scores = jnp.dot(xf, g_ref[...], preferred_element_type=jnp.float32)
    sig = jax.nn.sigmoid(scores)
    lane = lax.broadcasted_iota(jnp.int32, (NT, E), 1)
    biased = sig + b_ref[0, :][None, :]
    m1 = jnp.max(biased, axis=1, keepdims=True)
    idx1 = jnp.min(jnp.where(biased == m1, lane, E), axis=1, keepdims=True)
    sel1 = lane == idx1
    b2 = jnp.where(sel1, NEG, biased)
    m2 = jnp.max(b2, axis=1, keepdims=True)
    idx2 = jnp.min(jnp.where(b2 == m2, lane, E), axis=1, keepdims=True)
    sel2 = lane == idx2
    s0_ref[...] = jnp.sum(jnp.where(sel1, sig, 0.0), axis=1, keepdims=True)
    s1_ref[...] = jnp.sum(jnp.where(sel2, sig, 0.0), axis=1, keepdims=True)

    c = jnp.where(sel1 | sel2, 1.0, 0.0)
    ii = lax.broadcasted_iota(jnp.int32, (LANES, LANES), 0)
    jj = lax.broadcasted_iota(jnp.int32, (LANES, LANES), 1)
    lt = jnp.where(jj < ii, 1.0, 0.0)
    nch = NT // LANES
    base = jnp.zeros((1, E), jnp.float32)
    excl_chunks = []
    for g in range(nch):
        cg = lax.slice(c, (g * LANES, 0), ((g + 1) * LANES, E))
        eg = jnp.dot(lt, cg, preferred_element_type=jnp.float32)
        excl_chunks.append(eg + base)
        base = base + jnp.sum(cg, axis=0, keepdims=True)
    excl = jnp.concatenate(excl_chunks, axis=0)
    counts = base

    pc = jnp.floor((counts + (BLK - 1)) * (1.0 / BLK)) * BLK
    ii8 = lax.broadcasted_iota(jnp.int32, (E, E), 0)
    jj8 = lax.broadcasted_iota(jnp.int32, (E, E), 1)
    su = jnp.where(ii8 < jj8, 1.0, 0.0)
    offs = jnp.dot(pc, su, preferred_element_type=jnp.float32)
    dest = offs + excl
    d0_ref[...] = jnp.sum(jnp.where(sel1, dest, 0.0), axis=1,
                          keepdims=True).astype(jnp.int32)
    d1_ref[...] = jnp.sum(jnp.where(sel2, dest, 0.0), axis=1,
                          keepdims=True).astype(jnp.int32)

    cum_incl = offs + pc
    bcol = lax.broadcasted_iota(jnp.int32, (LANES, 1), 0).astype(jnp.float32)
    cmp = jnp.where(cum_incl * (1.0 / BLK) <= bcol, 1.0, 0.0)
    be = jnp.minimum(jnp.sum(cmp, axis=1, keepdims=True), float(E - 1))
    be_ref[...] = be.astype(jnp.int32)
    tp = jnp.sum(pc, axis=1, keepdims=True)
    act_ref[...] = jnp.where(bcol * BLK < tp, 1, 0).astype(jnp.int32)


def _run_router(xf, gate, expert_bias):
    bp = expert_bias.reshape(1, E)
    outs = pl.pallas_call(
        _router_body,
        grid=(1,),
        in_specs=[
            pl.BlockSpec((NT, DIM), lambda i: (0, 0)),
            pl.BlockSpec((DIM, E), lambda i: (0, 0)),
            pl.BlockSpec((1, E), lambda i: (0, 0)),
        ],
        out_specs=[
            pl.BlockSpec((NT, 1), lambda i: (0, 0)),
            pl.BlockSpec((NT, 1), lambda i: (0, 0)),
            pl.BlockSpec((NT, 1), lambda i: (0, 0)),
            pl.BlockSpec((NT, 1), lambda i: (0, 0)),
            pl.BlockSpec((LANES, 1), lambda i: (0, 0)),
            pl.BlockSpec((LANES, 1), lambda i: (0, 0)),
        ],
        out_shape=[
            jax.ShapeDtypeStruct((NT, 1), jnp.int32),
            jax.ShapeDtypeStruct((NT, 1), jnp.int32),
            jax.ShapeDtypeStruct((NT, 1), jnp.float32),
            jax.ShapeDtypeStruct((NT, 1), jnp.float32),
            jax.ShapeDtypeStruct((LANES, 1), jnp.int32),
            jax.ShapeDtypeStruct((LANES, 1), jnp.int32),
        ],
    )(xf, gate, bp)
    return outs


def _dispatch_body(x_hbm, d0_hbm, d1_hbm, xg_hbm, rows_v, d0_v, d1_v, sem):
    wid = lax.axis_index("s") * 2 + lax.axis_index("c")
    base = wid * TPW
    l0 = pltpu.async_copy(d0_hbm.at[pl.ds(base, TPW)], d0_v, sem)
    l1 = pltpu.async_copy(d1_hbm.at[pl.ds(base, TPW)], d1_v, sem)
    l2 = pltpu.async_copy(x_hbm.at[pl.ds(base, TPW)], rows_v, sem)
    l0.wait()
    l1.wait()
    l2.wait()
    c0 = pltpu.async_copy(rows_v, xg_hbm.at[d0_v], sem)
    c1 = pltpu.async_copy(rows_v, xg_hbm.at[d1_v], sem)
    c0.wait()
    c1.wait()


@functools.cache
def _build_dispatch():
    return pl.kernel(
        _dispatch_body,
        out_type=jax.ShapeDtypeStruct((RPAD, DIM), jnp.float32),
        mesh=plsc.VectorSubcoreMesh(core_axis_name="c", subcore_axis_name="s"),
        scratch_types=[
            pltpu.VMEM((TPW, DIM), jnp.float32),
            pltpu.VMEM((TPW,), jnp.int32),
            pltpu.VMEM((TPW,), jnp.int32),
            pltpu.SemaphoreType.DMA,
        ],
    )


def _dispatch(xf, d0, d1):
    return _build_dispatch()(xf, d0, d1)


def _gmm_body(be_ref, act_ref, na_ref, xg_ref, w1_ref, w3_ref, w2_ref,
              eo_ref, w1s, w3s, w2s):
    b = pl.program_id(0)
    prev = be_ref[jnp.maximum(b - 1, 0)]
    changed = (b == 0) | (be_ref[b] != prev)

    @pl.when((act_ref[b] == 1) & changed)
    def _():
        w1s[...] = w1_ref[0].astype(jnp.bfloat16)
        w3s[...] = w3_ref[0].astype(jnp.bfloat16)
        w2s[...] = w2_ref[0].astype(jnp.bfloat16)

    @pl.when(act_ref[b] == 1)
    def _():
        xb = xg_ref[...].astype(jnp.bfloat16)
        t1 = lax.dot_general(xb, w1s[...], _DN,
                             preferred_element_type=jnp.float32)
        t3 = lax.dot_general(xb, w3s[...], _DN,
                             preferred_element_type=jnp.float32)
        h = ((t1 * jax.nn.sigmoid(t1)) * t3).astype(jnp.bfloat16)
        eo_ref[...] = lax.dot_general(
            h, w2s[...], _DN, preferred_element_type=jnp.float32)


def _run_gmm(be, act, na, xg, w1, w3, w2):
    return pl.pallas_call(
        _gmm_body,
        grid_spec=pltpu.PrefetchScalarGridSpec(
            num_scalar_prefetch=3,
            grid=(NBLK,),
            in_specs=[
                pl.BlockSpec(
                    (BLK, DIM),
                    lambda b, be_r, act_r, na_r:
                        (jnp.minimum(b, na_r[0] - 1), 0)),
                pl.BlockSpec((1, HIDDEN, DIM),
                             lambda b, be_r, act_r, na_r: (be_r[b], 0, 0)),
                pl.BlockSpec((1, HIDDEN, DIM),
                             lambda b, be_r, act_r, na_r: (be_r[b], 0, 0)),
                pl.BlockSpec((1, DIM, HIDDEN),
                             lambda b, be_r, act_r, na_r: (be_r[b], 0, 0)),
            ],
            out_specs=pl.BlockSpec(
                (BLK, DIM),
                lambda b, be_r, act_r, na_r:
                    (jnp.minimum(b, na_r[0] - 1), 0)),
            scratch_shapes=[
                pltpu.VMEM((HIDDEN, DIM), jnp.bfloat16),
                pltpu.VMEM((HIDDEN, DIM), jnp.bfloat16),
                pltpu.VMEM((DIM, HIDDEN), jnp.bfloat16),
            ],
        ),
        out_shape=jax.ShapeDtypeStruct((RPAD, DIM), jnp.float32),
        compiler_params=pltpu.CompilerParams(
            vmem_limit_bytes=100 * 1024 * 1024,
        ),
    )(be, act, na, xg, w1, w3, w2)


def _collect_body(eo_hbm, d0_hbm, d1_hbm, g0_hbm, g1_hbm,
                  r0_v, r1_v, d0_v, d1_v, sem):
    wid = lax.axis_index("s") * 2 + lax.axis_index("c")
    base = wid * TPW
    half = TPW // 2
    l0 = pltpu.async_copy(d0_hbm.at[pl.ds(base, TPW)], d0_v, sem)
    l1 = pltpu.async_copy(d1_hbm.at[pl.ds(base, TPW)], d1_v, sem)
    l0.wait()
    l1.wait()
    for p in range(2):
        off = p * half
        g0 = pltpu.async_copy(eo_hbm.at[d0_v.at[pl.ds(off, half)]], r0_v, sem)
        g1 = pltpu.async_copy(eo_hbm.at[d1_v.at[pl.ds(off, half)]], r1_v, sem)
        g0.wait()
        g1.wait()
        s0 = pltpu.async_copy(r0_v, g0_hbm.at[pl.ds(base + off, half)], sem)
        s1 = pltpu.async_copy(r1_v, g1_hbm.at[pl.ds(base + off, half)], sem)
        s0.wait()
        s1.wait()


@functools.cache
def _build_collect():
    return pl.kernel(
        _collect_body,
        out_type=(jax.ShapeDtypeStruct((NT, DIM), jnp.float32),
                  jax.ShapeDtypeStruct((NT, DIM), jnp.float32)),
        mesh=plsc.VectorSubcoreMesh(core_axis_name="c", subcore_axis_name="s"),
        scratch_types=[
            pltpu.VMEM((TPW // 2, DIM), jnp.float32),
            pltpu.VMEM((TPW // 2, DIM), jnp.float32),
            pltpu.VMEM((TPW,), jnp.int32),
            pltpu.VMEM((TPW,), jnp.int32),
            pltpu.SemaphoreType.DMA,
        ],
    )


def _collect(eo, d0, d1):
    return _build_collect()(eo, d0, d1)


def _shared_body(xf_ref, ws1_ref, ws2_ref, ws3_ref, sh_ref, ws1s, ws2s, ws3s):
    i = pl.program_id(0)

    @pl.when(i == 0)
    def _():
        ws1s[...] = ws1_ref[...].astype(jnp.bfloat16)
        ws2s[...] = ws2_ref[...].astype(jnp.bfloat16)
        ws3s[...] = ws3_ref[...].astype(jnp.bfloat16)

    xb = xf_ref[...].astype(jnp.bfloat16)
    u1 = lax.dot_general(xb, ws1s[...], _DN,
                         preferred_element_type=jnp.float32)
    u3 = lax.dot_general(xb, ws3s[...], _DN,
                         preferred_element_type=jnp.float32)
    hs = ((u1 * jax.nn.sigmoid(u1)) * u3).astype(jnp.bfloat16)
    sh_ref[...] = lax.dot_general(hs, ws2s[...], _DN,
                                  preferred_element_type=jnp.float32)


def _run_shared(xf, ws1, ws2, ws3):
    nb = 4
    bt = NT // nb
    return pl.pallas_call(
        _shared_body,
        grid=(nb,),
        in_specs=[
            pl.BlockSpec((bt, DIM), lambda i: (i, 0)),
            pl.BlockSpec((HIDDEN, DIM), lambda i: (0, 0)),
            pl.BlockSpec((DIM, HIDDEN), lambda i: (0, 0)),
            pl.BlockSpec((HIDDEN, DIM), lambda i: (0, 0)),
        ],
        out_specs=pl.BlockSpec((bt, DIM), lambda i: (i, 0)),
        out_shape=jax.ShapeDtypeStruct((NT, DIM), jnp.float32),
        scratch_shapes=[
            pltpu.VMEM((HIDDEN, DIM), jnp.bfloat16),
            pltpu.VMEM((DIM, HIDDEN), jnp.bfloat16),
            pltpu.VMEM((HIDDEN, DIM), jnp.bfloat16),
        ],
    )(xf, ws1, ws2, ws3)


def _combine_body(sh_ref, g0_ref, g1_ref, s0_ref, s1_ref, out_ref):
    out_ref[...] = (s0_ref[...] * g0_ref[...]
                    + s1_ref[...] * g1_ref[...] + sh_ref[...])


def _run_combine(sh, g0, g1, s0, s1):
    nb = 4
    bt = NT // nb
    return pl.pallas_call(
        _combine_body,
        grid=(nb,),
        in_specs=[
            pl.BlockSpec((bt, DIM), lambda i: (i, 0)),
            pl.BlockSpec((bt, DIM), lambda i: (i, 0)),
            pl.BlockSpec((bt, DIM), lambda i: (i, 0)),
            pl.BlockSpec((bt, 1), lambda i: (i, 0)),
            pl.BlockSpec((bt, 1), lambda i: (i, 0)),
        ],
        out_specs=pl.BlockSpec((bt, DIM), lambda i: (i, 0)),
        out_shape=jax.ShapeDtypeStruct((NT, DIM), jnp.float32),
    )(sh, g0, g1, s0, s1)


def kernel(x, gate, w1, w2, w3, ws1, ws2, ws3, expert_bias):
    ob, ib, slen, dim = x.shape
    xf = x.reshape(NT, DIM)

    d0_2d, d1_2d, s0, s1, be_2d, act_2d = _run_router(xf, gate, expert_bias)
    d0 = d0_2d.reshape(NT)
    d1 = d1_2d.reshape(NT)
    be = be_2d.reshape(LANES)[:NBLK]
    act = act_2d.reshape(LANES)[:NBLK]

    xg = _dispatch(xf, d0, d1)

    na = jnp.maximum(jnp.sum(act), 1)
    eo = _run_gmm(be, act, na.reshape(1), xg, w1, w3, w2)

    sh = _run_shared(xf, ws1, ws2, ws3)
    g0, g1 = _collect(eo, d0, d1)

    out = _run_combine(sh, g0, g1, s0, s1)
    return out.reshape(ob, ib, slen, dim)


# speedup vs baseline: 1.1351x; 1.0622x over previous
import functools

import jax
import jax.numpy as jnp
from jax import lax
from jax.experimental import pallas as pl
from jax.experimental.pallas import tpu as pltpu
from jax.experimental.pallas import tpu_sc as plsc

DIM = 1024
HIDDEN = 1024
E = 8
TOP_K = 2
NT = 2048
LANES = 128
NEG = float("-inf")

BLK = 576
NBLK = (NT * TOP_K - E) // BLK + E
RPAD = NBLK * BLK

NWORKERS = 32
TPW = NT // NWORKERS
SL = DIM // LANES

_DN = (((1,), (1,)), ((), ()))


def _router_body(xf_ref, g_ref, b_ref, d0_ref, d1_ref, s0_ref,
                 s1_ref, be_ref, act_ref):
    xf = xf_ref[...]
    scores = jnp.dot(xf, g_ref[...], preferred_element_type=jnp.float32)
    sig = jax.nn.sigmoid(scores)
    lane = lax.broadcasted_iota(jnp.int32, (NT, E), 1)
    biased = sig + b_ref[0, :][None, :]
    m1 = jnp.max(biased, axis=1, keepdims=True)
    idx1 = jnp.min(jnp.where(biased == m1, lane, E), axis=1, keepdims=True)
    sel1 = lane == idx1
    b2 = jnp.where(sel1, NEG, biased)
    m2 = jnp.max(b2, axis=1, keepdims=True)
    idx2 = jnp.min(jnp.where(b2 == m2, lane, E), axis=1, keepdims=True)
    sel2 = lane == idx2
    s0_ref[...] = jnp.sum(jnp.where(sel1, sig, 0.0), axis=1, keepdims=True)
    s1_ref[...] = jnp.sum(jnp.where(sel2, sig, 0.0), axis=1, keepdims=True)

    c = jnp.where(sel1 | sel2, 1.0, 0.0)
    ii = lax.broadcasted_iota(jnp.int32, (LANES, LANES), 0)
    jj = lax.broadcasted_iota(jnp.int32, (LANES, LANES), 1)
    lt = jnp.where(jj < ii, 1.0, 0.0)
    nch = NT // LANES
    base = jnp.zeros((1, E), jnp.float32)
    excl_chunks = []
    for g in range(nch):
        cg = lax.slice(c, (g * LANES, 0), ((g + 1) * LANES, E))
        eg = jnp.dot(lt, cg, preferred_element_type=jnp.float32)
        excl_chunks.append(eg + base)
        base = base + jnp.sum(cg, axis=0, keepdims=True)
    excl = jnp.concatenate(excl_chunks, axis=0)
    counts = base

    pc = jnp.floor((counts + (BLK - 1)) * (1.0 / BLK)) * BLK
    ii8 = lax.broadcasted_iota(jnp.int32, (E, E), 0)
    jj8 = lax.broadcasted_iota(jnp.int32, (E, E), 1)
    su = jnp.where(ii8 < jj8, 1.0, 0.0)
    offs = jnp.dot(pc, su, preferred_element_type=jnp.float32)
    dest = offs + excl
    d0_ref[...] = jnp.sum(jnp.where(sel1, dest, 0.0), axis=1,
                          keepdims=True).astype(jnp.int32)
    d1_ref[...] = jnp.sum(jnp.where(sel2, dest, 0.0), axis=1,
                          keepdims=True).astype(jnp.int32)

    cum_incl = offs + pc
    bcol = lax.broadcasted_iota(jnp.int32, (LANES, 1), 0).astype(jnp.float32)
    cmp = jnp.where(cum_incl * (1.0 / BLK) <= bcol, 1.0, 0.0)
    be = jnp.minimum(jnp.sum(cmp, axis=1, keepdims=True), float(E - 1))
    be_ref[...] = be.astype(jnp.int32)
    tp = jnp.sum(pc, axis=1, keepdims=True)
    act_ref[...] = jnp.where(bcol * BLK < tp, 1, 0).astype(jnp.int32)


def _run_router(xf, gate, expert_bias):
    bp = expert_bias.reshape(1, E)
    outs = pl.pallas_call(
        _router_body,
        grid=(1,),
        in_specs=[
            pl.BlockSpec((NT, DIM), lambda i: (0, 0)),
            pl.BlockSpec((DIM, E), lambda i: (0, 0)),
            pl.BlockSpec((1, E), lambda i: (0, 0)),
        ],
        out_specs=[
            pl.BlockSpec((NT, 1), lambda i: (0, 0)),
            pl.BlockSpec((NT, 1), lambda i: (0, 0)),
            pl.BlockSpec((NT, 1), lambda i: (0, 0)),
            pl.BlockSpec((NT, 1), lambda i: (0, 0)),
            pl.BlockSpec((LANES, 1), lambda i: (0, 0)),
            pl.BlockSpec((LANES, 1), lambda i: (0, 0)),
        ],
        out_shape=[
            jax.ShapeDtypeStruct((NT, 1), jnp.int32),
            jax.ShapeDtypeStruct((NT, 1), jnp.int32),
            jax.ShapeDtypeStruct((NT, 1), jnp.float32),
            jax.ShapeDtypeStruct((NT, 1), jnp.float32),
            jax.ShapeDtypeStruct((LANES, 1), jnp.int32),
            jax.ShapeDtypeStruct((LANES, 1), jnp.int32),
        ],
    )(xf, gate, bp)
    return outs


def _dispatch_body(x_hbm, d0_hbm, d1_hbm, xg_hbm, rows_v, d0_v, d1_v, sem):
    wid = lax.axis_index("s") * 2 + lax.axis_index("c")
    base = wid * TPW
    l0 = pltpu.async_copy(d0_hbm.at[pl.ds(base, TPW)], d0_v, sem)
    l1 = pltpu.async_copy(d1_hbm.at[pl.ds(base, TPW)], d1_v, sem)
    l2 = pltpu.async_copy(x_hbm.at[pl.ds(base, TPW)], rows_v, sem)
    l0.wait()
    l1.wait()
    l2.wait()
    c0 = pltpu.async_copy(rows_v, xg_hbm.at[d0_v], sem)
    c1 = pltpu.async_copy(rows_v, xg_hbm.at[d1_v], sem)
    c0.wait()
    c1.wait()


@functools.cache
def _build_dispatch():
    return pl.kernel(
        _dispatch_body,
        out_type=jax.ShapeDtypeStruct((RPAD, DIM), jnp.float32),
        mesh=plsc.VectorSubcoreMesh(core_axis_name="c", subcore_axis_name="s"),
        scratch_types=[
            pltpu.VMEM((TPW, DIM), jnp.float32),
            pltpu.VMEM((TPW,), jnp.int32),
            pltpu.VMEM((TPW,), jnp.int32),
            pltpu.SemaphoreType.DMA,
        ],
    )


def _dispatch(xf, d0, d1):
    return _build_dispatch()(xf, d0, d1)


def _gmm_body(be_ref, act_ref, na_ref, xg_ref, w1_ref, w3_ref, w2_ref,
              eo_ref, w1s, w3s, w2s):
    b = pl.program_id(0)
    prev = be_ref[jnp.maximum(b - 1, 0)]
    changed = (b == 0) | (be_ref[b] != prev)

    @pl.when((act_ref[b] == 1) & changed)
    def _():
        w1s[...] = w1_ref[0].astype(jnp.bfloat16)
        w3s[...] = w3_ref[0].astype(jnp.bfloat16)
        w2s[...] = w2_ref[0].astype(jnp.bfloat16)

    @pl.when(act_ref[b] == 1)
    def _():
        xb = xg_ref[...].astype(jnp.bfloat16)
        t1 = lax.dot_general(xb, w1s[...], _DN,
                             preferred_element_type=jnp.float32)
        t3 = lax.dot_general(xb, w3s[...], _DN,
                             preferred_element_type=jnp.float32)
        h = ((t1 * jax.nn.sigmoid(t1)) * t3).astype(jnp.bfloat16)
        eo_ref[...] = lax.dot_general(
            h, w2s[...], _DN, preferred_element_type=jnp.float32)


def _run_gmm(be, act, na, xg, w1, w3, w2):
    return pl.pallas_call(
        _gmm_body,
        grid_spec=pltpu.PrefetchScalarGridSpec(
            num_scalar_prefetch=3,
            grid=(NBLK,),
            in_specs=[
                pl.BlockSpec(
                    (BLK, DIM),
                    lambda b, be_r, act_r, na_r:
                        (jnp.minimum(b, na_r[0] - 1), 0)),
                pl.BlockSpec((1, HIDDEN, DIM),
                             lambda b, be_r, act_r, na_r: (be_r[b], 0, 0)),
                pl.BlockSpec((1, HIDDEN, DIM),
                             lambda b, be_r, act_r, na_r: (be_r[b], 0, 0)),
                pl.BlockSpec((1, DIM, HIDDEN),
                             lambda b, be_r, act_r, na_r: (be_r[b], 0, 0)),
            ],
            out_specs=pl.BlockSpec(
                (BLK, DIM),
                lambda b, be_r, act_r, na_r:
                    (jnp.minimum(b, na_r[0] - 1), 0)),
            scratch_shapes=[
                pltpu.VMEM((HIDDEN, DIM), jnp.bfloat16),
                pltpu.VMEM((HIDDEN, DIM), jnp.bfloat16),
                pltpu.VMEM((DIM, HIDDEN), jnp.bfloat16),
            ],
        ),
        out_shape=jax.ShapeDtypeStruct((RPAD, DIM), jnp.float32),
        compiler_params=pltpu.CompilerParams(
            vmem_limit_bytes=100 * 1024 * 1024,
        ),
    )(be, act, na, xg, w1, w3, w2)


def _collect_body(eo_hbm, d0_hbm, d1_hbm, g0_hbm, g1_hbm,
                  r0_v, r1_v, d0_v, d1_v, sem):
    wid = lax.axis_index("s") * 2 + lax.axis_index("c")
    base = wid * TPW
    half = TPW // 2
    l0 = pltpu.async_copy(d0_hbm.at[pl.ds(base, TPW)], d0_v, sem)
    l1 = pltpu.async_copy(d1_hbm.at[pl.ds(base, TPW)], d1_v, sem)
    l0.wait()
    l1.wait()
    for p in range(2):
        off = p * half
        g0 = pltpu.async_copy(eo_hbm.at[d0_v.at[pl.ds(off, half)]], r0_v, sem)
        g1 = pltpu.async_copy(eo_hbm.at[d1_v.at[pl.ds(off, half)]], r1_v, sem)
        g0.wait()
        g1.wait()
        s0 = pltpu.async_copy(r0_v, g0_hbm.at[pl.ds(base + off, half)], sem)
        s1 = pltpu.async_copy(r1_v, g1_hbm.at[pl.ds(base + off, half)], sem)
        s0.wait()
        s1.wait()


@functools.cache
def _build_collect():
    return pl.kernel(
        _collect_body,
        out_type=(jax.ShapeDtypeStruct((NT, DIM), jnp.float32),
                  jax.ShapeDtypeStruct((NT, DIM), jnp.float32)),
        mesh=plsc.VectorSubcoreMesh(core_axis_name="c", subcore_axis_name="s"),
        scratch_types=[
            pltpu.VMEM((TPW // 2, DIM), jnp.float32),
            pltpu.VMEM((TPW // 2, DIM), jnp.float32),
            pltpu.VMEM((TPW,), jnp.int32),
            pltpu.VMEM((TPW,), jnp.int32),
            pltpu.SemaphoreType.DMA,
        ],
    )


def _collect(eo, d0, d1):
    return _build_collect()(eo, d0, d1)


def _shared_body(xf_ref, ws1_ref, ws2_ref, ws3_ref, sh_ref, ws1s, ws2s, ws3s):
    i = pl.program_id(0)

    @pl.when(i == 0)
    def _():
        ws1s[...] = ws1_ref[...].astype(jnp.bfloat16)
        ws2s[...] = ws2_ref[...].astype(jnp.bfloat16)
        ws3s[...] = ws3_ref[...].astype(jnp.bfloat16)

    xb = xf_ref[...].astype(jnp.bfloat16)
    u1 = lax.dot_general(xb, ws1s[...], _DN,
                         preferred_element_type=jnp.float32)
    u3 = lax.dot_general(xb, ws3s[...], _DN,
                         preferred_element_type=jnp.float32)
    hs = ((u1 * jax.nn.sigmoid(u1)) * u3).astype(jnp.bfloat16)
    sh_ref[...] = lax.dot_general(hs, ws2s[...], _DN,
                                  preferred_element_type=jnp.float32)


def _run_shared(xf, ws1, ws2, ws3):
    nb = 4
    bt = NT // nb
    return pl.pallas_call(
        _shared_body,
        grid=(nb,),
        in_specs=[
            pl.BlockSpec((bt, DIM), lambda i: (i, 0)),
            pl.BlockSpec((HIDDEN, DIM), lambda i: (0, 0)),
            pl.BlockSpec((DIM, HIDDEN), lambda i: (0, 0)),
            pl.BlockSpec((HIDDEN, DIM), lambda i: (0, 0)),
        ],
        out_specs=pl.BlockSpec((bt, DIM), lambda i: (i, 0)),
        out_shape=jax.ShapeDtypeStruct((NT, DIM), jnp.float32),
        scratch_shapes=[
            pltpu.VMEM((HIDDEN, DIM), jnp.bfloat16),
            pltpu.VMEM((DIM, HIDDEN), jnp.bfloat16),
            pltpu.VMEM((HIDDEN, DIM), jnp.bfloat16),
        ],
    )(xf, ws1, ws2, ws3)


def _combine_body(sh_ref, g0_ref, g1_ref, s0_ref, s1_ref, out_ref):
    out_ref[...] = (s0_ref[...] * g0_ref[...]
                    + s1_ref[...] * g1_ref[...] + sh_ref[...])


def _run_combine(sh, g0, g1, s0, s1):
    nb = 4
    bt = NT // nb
    return pl.pallas_call(
        _combine_body,
        grid=(nb,),
        in_specs=[
            pl.BlockSpec((bt, DIM), lambda i: (i, 0)),
            pl.BlockSpec((bt, DIM), lambda i: (i, 0)),
            pl.BlockSpec((bt, DIM), lambda i: (i, 0)),
            pl.BlockSpec((bt, 1), lambda i: (i, 0)),
            pl.BlockSpec((bt, 1), lambda i: (i, 0)),
        ],
        out_specs=pl.BlockSpec((bt, DIM), lambda i: (i, 0)),
        out_shape=jax.ShapeDtypeStruct((NT, DIM), jnp.float32),
    )(sh, g0, g1, s0, s1)


def kernel(x, gate, w1, w2, w3, ws1, ws2, ws3, expert_bias):
    ob, ib, slen, dim = x.shape
    xf = x.reshape(NT, DIM)

    d0_2d, d1_2d, s0, s1, be_2d, act_2d = _run_router(xf, gate, expert_bias)
    d0 = d0_2d.reshape(NT)
    d1 = d1_2d.reshape(NT)
    be = be_2d.reshape(LANES)[:NBLK]
    act = act_2d.reshape(LANES)[:NBLK]

    xg = _dispatch(xf, d0, d1)

    na = jnp.maximum(jnp.sum(act), 1)
    eo = _run_gmm(be, act, na.reshape(1), xg, w1, w3, w2)

    sh = _run_shared(xf, ws1, ws2, ws3)
    g0, g1 = _collect(eo, d0, d1)

    out = _run_combine(sh, g0, g1, s0, s1)
    return out.reshape(ob, ib, slen, dim)
